# Initial kernel scaffold; baseline (speedup 1.0000x reference)
import jax
import jax.numpy as jnp
from jax.experimental import pallas as pl


def kernel(ast_x, ast_edge_index, cfg_x, cfg_edge_index, pdg_x, pdg_edge_index, pdg_edge_type, source_code, params):
    raise NotImplementedError("write your pallas kernel here")


# speedup vs baseline: 2.0139x; 2.0139x over previous
import functools
import math

import jax
import jax.numpy as jnp
from jax import lax
from jax.experimental import pallas as pl
from jax.experimental.pallas import tpu as pltpu

N0 = 10000
E0 = 160000
NPAD = 10240
TRASH = N0
BM = 256


def _leaky(x):
    return jnp.where(x >= 0, x, 0.01 * x)


def _linear_body(*refs, nparts, act):
    out_ref = refs[-1]
    bias_ref = refs[-2]
    acc = jnp.zeros(out_ref.shape, jnp.float32)
    for i in range(nparts):
        a = refs[2 * i][...]
        w = refs[2 * i + 1][...]
        acc += jnp.dot(a, w, preferred_element_type=jnp.float32)
    acc = acc + bias_ref[...]
    if act:
        acc = _leaky(acc)
    out_ref[...] = acc


def _linear(parts, bias, act):
    rows = parts[0][0].shape[0]
    dout = parts[0][1].shape[1]
    bm = BM if rows % BM == 0 else rows
    grid = (rows // bm,)
    in_specs = []
    args = []
    for a, w in parts:
        di = a.shape[1]
        in_specs.append(pl.BlockSpec((bm, di), lambda i: (i, 0)))
        in_specs.append(pl.BlockSpec((di, dout), lambda i: (0, 0)))
        args += [a, w]
    b2 = jnp.zeros((1, dout), jnp.float32) if bias is None else bias.reshape(1, dout)
    in_specs.append(pl.BlockSpec((1, dout), lambda i: (0, 0)))
    args.append(b2)
    return pl.pallas_call(
        functools.partial(_linear_body, nparts=len(parts), act=act),
        grid=grid,
        in_specs=in_specs,
        out_specs=pl.BlockSpec((bm, dout), lambda i: (i, 0)),
        out_shape=jax.ShapeDtypeStruct((rows, dout), jnp.float32),
    )(*args)


def _head_body(r1, r2, r3, sc, lw, lb, w1, b1, g1, e1, w2, b2, g2, e2,
               w3, b3, g3, e3, cw, cb, out):
    bnc = 1.0 / math.sqrt(1.0 + 1e-5)
    llm = jnp.dot(sc[...], lw[...], preferred_element_type=jnp.float32) + lb[...]
    h = jnp.concatenate([r1[...], r2[...], r3[...], llm], axis=1)
    h = jnp.dot(h, w1[...], preferred_element_type=jnp.float32) + b1[...]
    h = _leaky(g1[...] * h * bnc + e1[...])
    h = jnp.dot(h, w2[...], preferred_element_type=jnp.float32) + b2[...]
    h = _leaky(g2[...] * h * bnc + e2[...])
    h = jnp.dot(h, w3[...], preferred_element_type=jnp.float32) + b3[...]
    h = _leaky(g3[...] * h * bnc + e3[...])
    out[...] = jnp.dot(h, cw[...], preferred_element_type=jnp.float32) + cb[...]


def _head(r1, r2, r3, sc, params):
    m = params['mlp']
    lp = params['llm']
    cp = params['clf']
    args = [r1, r2, r3, sc,
            lp['W'], lp['b'].reshape(1, -1),
            m['W1'], m['b1'].reshape(1, -1), m['g1'].reshape(1, -1), m['be1'].reshape(1, -1),
            m['W2'], m['b2'].reshape(1, -1), m['g2'].reshape(1, -1), m['be2'].reshape(1, -1),
            m['W3'], m['b3'].reshape(1, -1), m['g3'].reshape(1, -1), m['be3'].reshape(1, -1),
            cp['W'], cp['b'].reshape(1, -1)]
    return pl.pallas_call(
        _head_body,
        out_shape=jax.ShapeDtypeStruct((1, 2), jnp.float32),
    )(*args)


def _seg_rows(y, srcg, dstp):
    return jnp.zeros((NPAD, y.shape[1]), jnp.float32).at[dstp].add(y[srcg])


def _seg_scalar(vals, dstp):
    return jnp.zeros((NPAD,), jnp.float32).at[dstp].add(vals)


def _degree(dstp):
    return _seg_scalar(jnp.ones((E0,), jnp.float32), dstp) + 1.0


def _pool(h, src, dstp, alive, pp, k):
    sr = _linear([(h, jnp.concatenate([pp['Wrel'], pp['Wroot']], axis=1))], None, False)
    aggs = _seg_scalar(sr[src, 0], dstp)
    score = aggs + pp['brel'][0] + sr[:, 1]
    smask = jnp.where(alive > 0, score, -jnp.inf)
    _, perm = lax.top_k(smask, k)
    sel = jnp.zeros((NPAD,), jnp.float32).at[perm].set(1.0)
    gfac = jnp.tanh(score) * sel
    return h * gfac[:, None], sel


def _attn(h, alive, gw, gb):
    gate = _linear([(h, gw)], gb, False)[:, 0]
    g = jnp.where(alive > 0, gate, -jnp.inf)
    e = jnp.exp(g - jnp.max(g)) * alive
    a = e / jnp.sum(e)
    return _linear([(a.reshape(1, NPAD), h)], None, False)


def _gcn_branch(x, src, dst, p):
    alive = (jnp.arange(NPAD) < N0).astype(jnp.float32)
    dstp = dst
    dis = lax.rsqrt(_degree(dstp))
    raw = _seg_rows(x * dis[:, None], src, dstp)
    z = raw * dis[:, None] + x * (dis * dis)[:, None]
    h = _linear([(z, p['W1'])], p['b1'], True)
    h, alive = _pool(h, src, dstp, alive, p['pool1'], 8000)
    ok = alive[src] * alive[dst]
    dstp = jnp.where(ok > 0, dst, TRASH)
    dis = lax.rsqrt(_degree(dstp))
    xw = _linear([(h, p['W2'])], None, False)
    raw = _seg_rows(xw * dis[:, None], src, dstp)
    h = _leaky(raw * dis[:, None] + xw * (dis * dis)[:, None] + p['b2'][None, :])
    h, alive = _pool(h, src, dstp, alive, p['pool2'], 6400)
    ok = alive[src] * alive[dst]
    dstp = jnp.where(ok > 0, dst, TRASH)
    dis = lax.rsqrt(_degree(dstp))
    xw = _linear([(h, p['W3'])], None, False)
    raw = _seg_rows(xw * dis[:, None], src, dstp)
    h = _leaky(raw * dis[:, None] + xw * (dis * dis)[:, None] + p['b3'][None, :])
    return _attn(h, alive, p['gate_W'], p['gate_b'])


def _rgcn_conv_pre(x, src, dstps, Wr, Wroot, b):
    parts = [(x, Wroot)]
    for r in range(2):
        raw = _seg_rows(x, src, dstps[r])
        cnt = _seg_scalar(jnp.ones((E0,), jnp.float32), dstps[r])
        parts.append((raw / jnp.maximum(cnt, 1.0)[:, None], Wr[r]))
    return _linear(parts, b, True)


def _rgcn_conv_post(h, src, dstps, Wr, Wroot, b):
    out = _linear([(h, Wroot)], b, False)
    for r in range(2):
        xw = _linear([(h, Wr[r])], None, False)
        raw = _seg_rows(xw, src, dstps[r])
        cnt = _seg_scalar(jnp.ones((E0,), jnp.float32), dstps[r])
        out = out + raw / jnp.maximum(cnt, 1.0)[:, None]
    return _leaky(out)


def _pdg_branch(x, src, dst, et, p):
    alive = (jnp.arange(NPAD) < N0).astype(jnp.float32)
    dstp = dst
    rel = [jnp.where(et == r, dst, TRASH) for r in range(2)]
    h = _rgcn_conv_pre(x, src, rel, p['Wr1'], p['Wroot1'], p['b1'])
    h, alive = _pool(h, src, dstp, alive, p['pool1'], 8000)
    ok = alive[src] * alive[dst]
    dstp = jnp.where(ok > 0, dst, TRASH)
    rel = [jnp.where((ok > 0) & (et == r), dst, TRASH) for r in range(2)]
    h = _rgcn_conv_post(h, src, rel, p['Wr2'], p['Wroot2'], p['b2'])
    h, alive = _pool(h, src, dstp, alive, p['pool2'], 6400)
    ok = alive[src] * alive[dst]
    dstp = jnp.where(ok > 0, dst, TRASH)
    rel = [jnp.where((ok > 0) & (et == r), dst, TRASH) for r in range(2)]
    h = _rgcn_conv_post(h, src, rel, p['Wr3'], p['Wroot3'], p['b3'])
    return _attn(h, alive, p['gate_W'], p['gate_b'])


def _pad_rows(x):
    return jnp.pad(x, ((0, NPAD - N0), (0, 0)))


def kernel(ast_x, ast_edge_index, cfg_x, cfg_edge_index, pdg_x,
           pdg_edge_index, pdg_edge_type, source_code, params):
    r1 = _gcn_branch(_pad_rows(ast_x), ast_edge_index[0], ast_edge_index[1],
                     params['ast'])
    r2 = _gcn_branch(_pad_rows(cfg_x), cfg_edge_index[0], cfg_edge_index[1],
                     params['cfg'])
    et = (pdg_edge_type != jnp.min(pdg_edge_type)).astype(jnp.int32)
    r3 = _pdg_branch(_pad_rows(pdg_x), pdg_edge_index[0], pdg_edge_index[1],
                     et, params['pdg'])
    return _head(r1, r2, r3, source_code, params)
